# baseline (speedup 1.0000x reference)
import functools

import jax
import jax.numpy as jnp
from jax import lax
from jax.experimental import pallas as pl
from jax.experimental.pallas import tpu as pltpu
from jax.experimental.pallas import tpu_sc as plsc

_B, _N, _M = 8, 20000, 100
_VPB = _N // 16
_MAIN = _VPB // 4
_NB = 8
_BV = _MAIN // _NB
_BLK = _BV * 16
_CHUNK = (_MAIN + 1) * 16
_TROWS = 112

_LN2 = 0.6931471805599453
_P5 = (0.04392863, -0.40947559, 1.61017755, -3.52021884, 5.06975632,
       -2.79415368)


def _ln(x):
    bits = lax.bitcast_convert_type(x, jnp.int32)
    e = ((bits >> 23) & 0xFF) - 127
    m = lax.bitcast_convert_type((bits & 0x007FFFFF) | 0x3F800000,
                                 jnp.float32)
    p = jnp.float32(_P5[0])
    for c in _P5[1:]:
        p = p * m + c
    return (e.astype(jnp.float32) + p) * _LN2


def _body(boxes_hbm, gt_hbm, flag_hbm, idx_hbm,
          cls_hbm, clsm_hbm, reg_hbm, regm_hbm,
          tbl_v, tt_v, x1_v, y1_v, x2_v, y2_v, flag_v, idx_v,
          cls_v, clsm_v, dx_v, dy_v, dw_v, dh_v, regm_v,
          stbl, s0, s1, s2, s3, s4, s5, s6, s7, sx, sout):
    nc = 2
    w = lax.axis_index("s") * nc + lax.axis_index("c")
    b = w // 4
    q = w % 4
    has_extra = q < 2
    n0 = (q * 313 - jnp.maximum(q - 2, 0)) * 16
    pbase = b * 4 * _N + n0
    abase = b * _N + n0
    n_main = _MAIN * 16

    def in_copies(off, length, sem):
        return [
            pltpu.make_async_copy(boxes_hbm.at[pl.ds(pbase + off, length)],
                                  x1_v.at[pl.ds(off, length)], sem),
            pltpu.make_async_copy(
                boxes_hbm.at[pl.ds(pbase + _N + off, length)],
                y1_v.at[pl.ds(off, length)], sem),
            pltpu.make_async_copy(
                boxes_hbm.at[pl.ds(pbase + 2 * _N + off, length)],
                x2_v.at[pl.ds(off, length)], sem),
            pltpu.make_async_copy(
                boxes_hbm.at[pl.ds(pbase + 3 * _N + off, length)],
                y2_v.at[pl.ds(off, length)], sem),
            pltpu.make_async_copy(flag_hbm.at[pl.ds(abase + off, length)],
                                  flag_v.at[pl.ds(off, length)], sem),
            pltpu.make_async_copy(idx_hbm.at[pl.ds(abase + off, length)],
                                  idx_v.at[pl.ds(off, length)], sem),
        ]

    def out_copies(off, length):
        return [
            pltpu.make_async_copy(dx_v.at[pl.ds(off, length)],
                                  reg_hbm.at[pl.ds(pbase + off, length)],
                                  sout),
            pltpu.make_async_copy(dy_v.at[pl.ds(off, length)],
                                  reg_hbm.at[pl.ds(pbase + _N + off, length)],
                                  sout),
            pltpu.make_async_copy(
                dw_v.at[pl.ds(off, length)],
                reg_hbm.at[pl.ds(pbase + 2 * _N + off, length)], sout),
            pltpu.make_async_copy(
                dh_v.at[pl.ds(off, length)],
                reg_hbm.at[pl.ds(pbase + 3 * _N + off, length)], sout),
            pltpu.make_async_copy(cls_v.at[pl.ds(off, length)],
                                  cls_hbm.at[pl.ds(abase + off, length)],
                                  sout),
            pltpu.make_async_copy(clsm_v.at[pl.ds(off, length)],
                                  clsm_hbm.at[pl.ds(abase + off, length)],
                                  sout),
            pltpu.make_async_copy(regm_v.at[pl.ds(off, length)],
                                  regm_hbm.at[pl.ds(abase + off, length)],
                                  sout),
        ]

    tbl_cp = pltpu.make_async_copy(gt_hbm, tbl_v, stbl)
    tbl_cp.start()
    sems = (s0, s1, s2, s3, s4, s5, s6, s7)
    for k in range(_NB):
        for c in in_copies(k * _BLK, _BLK, sems[k]):
            c.start()

    @pl.when(has_extra)
    def _():
        for c in in_copies(n_main, 16, sx):
            c.start()

    lanes = lax.iota(jnp.int32, 16)
    tbl_cp.wait()
    tbase = b * (_M * 5)
    for j in range(_TROWS // 16):
        r = jnp.minimum(j * 16 + lanes, _M - 1) * 5 + tbase
        gx1 = plsc.load_gather(tbl_v, [r])
        gy1 = plsc.load_gather(tbl_v, [r + 1])
        gx2 = plsc.load_gather(tbl_v, [r + 2])
        gy2 = plsc.load_gather(tbl_v, [r + 3])
        gcl = plsc.load_gather(tbl_v, [r + 4])
        sl = pl.ds(j * 16, 16)
        tt_v[sl] = gx1 + gx2
        tt_v[pl.ds(_TROWS + j * 16, 16)] = gy1 + gy2
        tt_v[pl.ds(2 * _TROWS + j * 16, 16)] = _ln(
            jnp.maximum(gx2 - gx1, 1e-3))
        tt_v[pl.ds(3 * _TROWS + j * 16, 16)] = _ln(
            jnp.maximum(gy2 - gy1, 1e-3))
        tt_v[pl.ds(4 * _TROWS + j * 16, 16)] = gcl

    def one_vec(i):
        al = i * 16
        sl = pl.ds(al, 16)
        idx16 = idx_v[sl]
        gsx = plsc.load_gather(tt_v, [idx16])
        gsy = plsc.load_gather(tt_v, [idx16 + _TROWS])
        lgw = plsc.load_gather(tt_v, [idx16 + 2 * _TROWS])
        lgh = plsc.load_gather(tt_v, [idx16 + 3 * _TROWS])
        gcl = plsc.load_gather(tt_v, [idx16 + 4 * _TROWS])

        x1 = x1_v[sl]
        y1 = y1_v[sl]
        x2 = x2_v[sl]
        y2 = y2_v[sl]
        bw = jnp.maximum(x2 - x1, 1e-3)
        bh = jnp.maximum(y2 - y1, 1e-3)
        dx_v[sl] = (gsx - (x1 + x2)) * 0.5 / bw
        dy_v[sl] = (gsy - (y1 + y2)) * 0.5 / bh
        dw_v[sl] = lgw - _ln(bw)
        dh_v[sl] = lgh - _ln(bh)

        flag = flag_v[sl]
        cls = jnp.where(flag == 0, 0.0, gcl)
        regm_v[sl] = jnp.where((flag > 0) & (cls > 0.0), 1, 0)
        cls = jnp.where(flag < 0, -jnp.abs(cls), cls)
        cls_v[sl] = cls
        clsm_v[sl] = jnp.where(cls >= 0.0, 1, 0)

    def loop_body(i, carry):
        one_vec(i)
        return carry

    for k in range(_NB):
        for c in in_copies(k * _BLK, _BLK, sems[k]):
            c.wait()
        lax.fori_loop(k * _BV, (k + 1) * _BV, loop_body, 0)
        for c in out_copies(k * _BLK, _BLK):
            c.start()

    @pl.when(has_extra)
    def _():
        for c in in_copies(n_main, 16, sx):
            c.wait()
        one_vec(jnp.int32(_MAIN))
        for c in out_copies(n_main, 16):
            c.start()

    for k in range(_NB):
        for c in out_copies(k * _BLK, _BLK):
            c.wait()

    @pl.when(has_extra)
    def _():
        for c in out_copies(n_main, 16):
            c.wait()


@jax.jit
def _run(boxes_pl, gt_flat, flag_f, idx_f):
    call = pl.kernel(
        _body,
        out_type=(
            jax.ShapeDtypeStruct((_B * _N,), jnp.float32),
            jax.ShapeDtypeStruct((_B * _N,), jnp.int32),
            jax.ShapeDtypeStruct((_B * 4 * _N,), jnp.float32),
            jax.ShapeDtypeStruct((_B * _N,), jnp.int32),
        ),
        mesh=plsc.VectorSubcoreMesh(core_axis_name="c", subcore_axis_name="s"),
        compiler_params=pltpu.CompilerParams(needs_layout_passes=False),
        scratch_types=[
            pltpu.VMEM((_B * _M * 5,), jnp.float32),
            pltpu.VMEM((5 * _TROWS,), jnp.float32),
            pltpu.VMEM((_CHUNK,), jnp.float32),
            pltpu.VMEM((_CHUNK,), jnp.float32),
            pltpu.VMEM((_CHUNK,), jnp.float32),
            pltpu.VMEM((_CHUNK,), jnp.float32),
            pltpu.VMEM((_CHUNK,), jnp.int32),
            pltpu.VMEM((_CHUNK,), jnp.int32),
            pltpu.VMEM((_CHUNK,), jnp.float32),
            pltpu.VMEM((_CHUNK,), jnp.int32),
            pltpu.VMEM((_CHUNK,), jnp.float32),
            pltpu.VMEM((_CHUNK,), jnp.float32),
            pltpu.VMEM((_CHUNK,), jnp.float32),
            pltpu.VMEM((_CHUNK,), jnp.float32),
            pltpu.VMEM((_CHUNK,), jnp.int32),
            pltpu.SemaphoreType.DMA,
            pltpu.SemaphoreType.DMA,
            pltpu.SemaphoreType.DMA,
            pltpu.SemaphoreType.DMA,
            pltpu.SemaphoreType.DMA,
            pltpu.SemaphoreType.DMA,
            pltpu.SemaphoreType.DMA,
            pltpu.SemaphoreType.DMA,
            pltpu.SemaphoreType.DMA,
            pltpu.SemaphoreType.DMA,
            pltpu.SemaphoreType.DMA,
        ],
    )
    return call(boxes_pl, gt_flat, flag_f, idx_f)


def kernel(boxes, gt_boxes, match_pos_flag, match_gt_id):
    boxes_pl = boxes.transpose(0, 2, 1).reshape(-1)
    cls, clsm, reg, regm = _run(boxes_pl, gt_boxes.reshape(-1),
                                match_pos_flag.reshape(-1),
                                match_gt_id.reshape(-1))
    cls_label = cls.reshape(_B, _N, 1)
    cls_label_mask = clsm.astype(jnp.bool_).reshape(_B, _N, 1)
    reg_label = reg.reshape(_B, 4, _N).transpose(0, 2, 1)
    reg_label_mask = jnp.broadcast_to(
        regm.astype(jnp.bool_).reshape(_B, _N, 1), (_B, _N, 4))
    return cls_label, cls_label_mask, reg_label, reg_label_mask


# speedup vs baseline: 1.0288x; 1.0288x over previous
import functools

import jax
import jax.numpy as jnp
from jax import lax
from jax.experimental import pallas as pl
from jax.experimental.pallas import tpu as pltpu
from jax.experimental.pallas import tpu_sc as plsc

_B, _N, _M = 8, 20000, 100
_VPB = _N // 16
_MAIN = _VPB // 4
_NB = 2
_BV = _MAIN // _NB
_BLK = _BV * 16
_CHUNK = (_MAIN + 1) * 16
_TROWS = 112

_LN2 = 0.6931471805599453
_P5 = (0.04392863, -0.40947559, 1.61017755, -3.52021884, 5.06975632,
       -2.79415368)


def _ln(x):
    bits = lax.bitcast_convert_type(x, jnp.int32)
    e = ((bits >> 23) & 0xFF) - 127
    m = lax.bitcast_convert_type((bits & 0x007FFFFF) | 0x3F800000,
                                 jnp.float32)
    p = jnp.float32(_P5[0])
    for c in _P5[1:]:
        p = p * m + c
    return (e.astype(jnp.float32) + p) * _LN2


def _body(boxes_hbm, gt_hbm, flag_hbm, idx_hbm,
          cls_hbm, clsm_hbm, reg_hbm, regm_hbm,
          tbl_v, tt_v, x1_v, y1_v, x2_v, y2_v, flag_v, idx_v,
          cls_v, clsm_v, dx_v, dy_v, dw_v, dh_v, regm_v,
          stbl, s0, s1, sx, sout):
    nc = 2
    w = lax.axis_index("s") * nc + lax.axis_index("c")
    b = w // 4
    q = w % 4
    has_extra = q < 2
    n0 = (q * 313 - jnp.maximum(q - 2, 0)) * 16
    pbase = b * 4 * _N + n0
    abase = b * _N + n0
    n_main = _MAIN * 16

    def in_copies(off, length, sem):
        return [
            pltpu.make_async_copy(boxes_hbm.at[pl.ds(pbase + off, length)],
                                  x1_v.at[pl.ds(off, length)], sem),
            pltpu.make_async_copy(
                boxes_hbm.at[pl.ds(pbase + _N + off, length)],
                y1_v.at[pl.ds(off, length)], sem),
            pltpu.make_async_copy(
                boxes_hbm.at[pl.ds(pbase + 2 * _N + off, length)],
                x2_v.at[pl.ds(off, length)], sem),
            pltpu.make_async_copy(
                boxes_hbm.at[pl.ds(pbase + 3 * _N + off, length)],
                y2_v.at[pl.ds(off, length)], sem),
            pltpu.make_async_copy(flag_hbm.at[pl.ds(abase + off, length)],
                                  flag_v.at[pl.ds(off, length)], sem),
            pltpu.make_async_copy(idx_hbm.at[pl.ds(abase + off, length)],
                                  idx_v.at[pl.ds(off, length)], sem),
        ]

    def out_copies(off, length):
        return [
            pltpu.make_async_copy(dx_v.at[pl.ds(off, length)],
                                  reg_hbm.at[pl.ds(pbase + off, length)],
                                  sout),
            pltpu.make_async_copy(dy_v.at[pl.ds(off, length)],
                                  reg_hbm.at[pl.ds(pbase + _N + off, length)],
                                  sout),
            pltpu.make_async_copy(
                dw_v.at[pl.ds(off, length)],
                reg_hbm.at[pl.ds(pbase + 2 * _N + off, length)], sout),
            pltpu.make_async_copy(
                dh_v.at[pl.ds(off, length)],
                reg_hbm.at[pl.ds(pbase + 3 * _N + off, length)], sout),
            pltpu.make_async_copy(cls_v.at[pl.ds(off, length)],
                                  cls_hbm.at[pl.ds(abase + off, length)],
                                  sout),
            pltpu.make_async_copy(clsm_v.at[pl.ds(off, length)],
                                  clsm_hbm.at[pl.ds(abase + off, length)],
                                  sout),
            pltpu.make_async_copy(regm_v.at[pl.ds(off, length)],
                                  regm_hbm.at[pl.ds(abase + off, length)],
                                  sout),
        ]

    tbl_cp = pltpu.make_async_copy(gt_hbm, tbl_v, stbl)
    tbl_cp.start()
    sems = (s0, s1)
    for k in range(_NB):
        for c in in_copies(k * _BLK, _BLK, sems[k]):
            c.start()

    @pl.when(has_extra)
    def _():
        for c in in_copies(n_main, 16, sx):
            c.start()

    lanes = lax.iota(jnp.int32, 16)
    tbl_cp.wait()
    tbase = b * (_M * 5)
    for j in range(_TROWS // 16):
        r = jnp.minimum(j * 16 + lanes, _M - 1) * 5 + tbase
        gx1 = plsc.load_gather(tbl_v, [r])
        gy1 = plsc.load_gather(tbl_v, [r + 1])
        gx2 = plsc.load_gather(tbl_v, [r + 2])
        gy2 = plsc.load_gather(tbl_v, [r + 3])
        gcl = plsc.load_gather(tbl_v, [r + 4])
        sl = pl.ds(j * 16, 16)
        tt_v[sl] = gx1 + gx2
        tt_v[pl.ds(_TROWS + j * 16, 16)] = gy1 + gy2
        tt_v[pl.ds(2 * _TROWS + j * 16, 16)] = _ln(
            jnp.maximum(gx2 - gx1, 1e-3))
        tt_v[pl.ds(3 * _TROWS + j * 16, 16)] = _ln(
            jnp.maximum(gy2 - gy1, 1e-3))
        tt_v[pl.ds(4 * _TROWS + j * 16, 16)] = gcl

    def one_vec(i):
        al = i * 16
        sl = pl.ds(al, 16)
        idx16 = idx_v[sl]
        gsx = plsc.load_gather(tt_v, [idx16])
        gsy = plsc.load_gather(tt_v, [idx16 + _TROWS])
        lgw = plsc.load_gather(tt_v, [idx16 + 2 * _TROWS])
        lgh = plsc.load_gather(tt_v, [idx16 + 3 * _TROWS])
        gcl = plsc.load_gather(tt_v, [idx16 + 4 * _TROWS])

        x1 = x1_v[sl]
        y1 = y1_v[sl]
        x2 = x2_v[sl]
        y2 = y2_v[sl]
        bw = jnp.maximum(x2 - x1, 1e-3)
        bh = jnp.maximum(y2 - y1, 1e-3)
        dx_v[sl] = (gsx - (x1 + x2)) * 0.5 / bw
        dy_v[sl] = (gsy - (y1 + y2)) * 0.5 / bh
        dw_v[sl] = lgw - _ln(bw)
        dh_v[sl] = lgh - _ln(bh)

        flag = flag_v[sl]
        cls = jnp.where(flag == 0, 0.0, gcl)
        regm_v[sl] = jnp.where((flag > 0) & (cls > 0.0), 1, 0)
        cls = jnp.where(flag < 0, -jnp.abs(cls), cls)
        cls_v[sl] = cls
        clsm_v[sl] = jnp.where(cls >= 0.0, 1, 0)

    def loop_body(i, carry):
        one_vec(i)
        return carry

    for k in range(_NB):
        for c in in_copies(k * _BLK, _BLK, sems[k]):
            c.wait()
        lax.fori_loop(k * _BV, (k + 1) * _BV, loop_body, 0)
        for c in out_copies(k * _BLK, _BLK):
            c.start()

    @pl.when(has_extra)
    def _():
        for c in in_copies(n_main, 16, sx):
            c.wait()
        one_vec(jnp.int32(_MAIN))
        for c in out_copies(n_main, 16):
            c.start()

    for k in range(_NB):
        for c in out_copies(k * _BLK, _BLK):
            c.wait()

    @pl.when(has_extra)
    def _():
        for c in out_copies(n_main, 16):
            c.wait()


@jax.jit
def _run(boxes_pl, gt_flat, flag_f, idx_f):
    call = pl.kernel(
        _body,
        out_type=(
            jax.ShapeDtypeStruct((_B * _N,), jnp.float32),
            jax.ShapeDtypeStruct((_B * _N,), jnp.int32),
            jax.ShapeDtypeStruct((_B * 4 * _N,), jnp.float32),
            jax.ShapeDtypeStruct((_B * _N,), jnp.int32),
        ),
        mesh=plsc.VectorSubcoreMesh(core_axis_name="c", subcore_axis_name="s"),
        compiler_params=pltpu.CompilerParams(needs_layout_passes=False),
        scratch_types=[
            pltpu.VMEM((_B * _M * 5,), jnp.float32),
            pltpu.VMEM((5 * _TROWS,), jnp.float32),
            pltpu.VMEM((_CHUNK,), jnp.float32),
            pltpu.VMEM((_CHUNK,), jnp.float32),
            pltpu.VMEM((_CHUNK,), jnp.float32),
            pltpu.VMEM((_CHUNK,), jnp.float32),
            pltpu.VMEM((_CHUNK,), jnp.int32),
            pltpu.VMEM((_CHUNK,), jnp.int32),
            pltpu.VMEM((_CHUNK,), jnp.float32),
            pltpu.VMEM((_CHUNK,), jnp.int32),
            pltpu.VMEM((_CHUNK,), jnp.float32),
            pltpu.VMEM((_CHUNK,), jnp.float32),
            pltpu.VMEM((_CHUNK,), jnp.float32),
            pltpu.VMEM((_CHUNK,), jnp.float32),
            pltpu.VMEM((_CHUNK,), jnp.int32),
            pltpu.SemaphoreType.DMA,
            pltpu.SemaphoreType.DMA,
            pltpu.SemaphoreType.DMA,
            pltpu.SemaphoreType.DMA,
            pltpu.SemaphoreType.DMA,
        ],
    )
    return call(boxes_pl, gt_flat, flag_f, idx_f)


def kernel(boxes, gt_boxes, match_pos_flag, match_gt_id):
    boxes_pl = boxes.transpose(0, 2, 1).reshape(-1)
    cls, clsm, reg, regm = _run(boxes_pl, gt_boxes.reshape(-1),
                                match_pos_flag.reshape(-1),
                                match_gt_id.reshape(-1))
    cls_label = cls.reshape(_B, _N, 1)
    cls_label_mask = clsm.astype(jnp.bool_).reshape(_B, _N, 1)
    reg_label = reg.reshape(_B, 4, _N).transpose(0, 2, 1)
    reg_label_mask = jnp.broadcast_to(
        regm.astype(jnp.bool_).reshape(_B, _N, 1), (_B, _N, 4))
    return cls_label, cls_label_mask, reg_label, reg_label_mask


# speedup vs baseline: 1.0294x; 1.0006x over previous
import functools

import jax
import jax.numpy as jnp
from jax import lax
from jax.experimental import pallas as pl
from jax.experimental.pallas import tpu as pltpu
from jax.experimental.pallas import tpu_sc as plsc

_B, _N, _M = 8, 20000, 100
_VPB = _N // 16
_MAIN = _VPB // 4
_NB = 2
_BV = _MAIN // _NB
_BLK = _BV * 16
_CHUNK = (_MAIN + 1) * 16
_TROWS = 112

_LN2 = 0.6931471805599453
_P5 = (0.04392863, -0.40947559, 1.61017755, -3.52021884, 5.06975632,
       -2.79415368)


def _ln(x):
    bits = lax.bitcast_convert_type(x, jnp.int32)
    e = ((bits >> 23) & 0xFF) - 127
    m = lax.bitcast_convert_type((bits & 0x007FFFFF) | 0x3F800000,
                                 jnp.float32)
    p = jnp.float32(_P5[0])
    for c in _P5[1:]:
        p = p * m + c
    return (e.astype(jnp.float32) + p) * _LN2


def _body(boxes_hbm, gt_hbm, flag_hbm, idx_hbm,
          cls_hbm, clsm_hbm, reg_hbm, regm_hbm,
          tbl_v, tt_v, x1_v, y1_v, x2_v, y2_v, flag_v, idx_v,
          cls_v, clsm_v, dx_v, dy_v, dw_v, dh_v, regm_v,
          stbl, s0, s1, sx, sout):
    nc = 2
    w = lax.axis_index("s") * nc + lax.axis_index("c")
    b = w // 4
    q = w % 4
    has_extra = q < 2
    n0 = (q * 313 - jnp.maximum(q - 2, 0)) * 16
    pbase = b * 4 * _N + n0
    abase = b * _N + n0
    n_main = _MAIN * 16

    def in_copies(off, length, sem):
        return [
            pltpu.make_async_copy(boxes_hbm.at[pl.ds(pbase + off, length)],
                                  x1_v.at[pl.ds(off, length)], sem),
            pltpu.make_async_copy(
                boxes_hbm.at[pl.ds(pbase + _N + off, length)],
                y1_v.at[pl.ds(off, length)], sem),
            pltpu.make_async_copy(
                boxes_hbm.at[pl.ds(pbase + 2 * _N + off, length)],
                x2_v.at[pl.ds(off, length)], sem),
            pltpu.make_async_copy(
                boxes_hbm.at[pl.ds(pbase + 3 * _N + off, length)],
                y2_v.at[pl.ds(off, length)], sem),
            pltpu.make_async_copy(flag_hbm.at[pl.ds(abase + off, length)],
                                  flag_v.at[pl.ds(off, length)], sem),
            pltpu.make_async_copy(idx_hbm.at[pl.ds(abase + off, length)],
                                  idx_v.at[pl.ds(off, length)], sem),
        ]

    def out_copies(off, length):
        return [
            pltpu.make_async_copy(dx_v.at[pl.ds(off, length)],
                                  reg_hbm.at[pl.ds(pbase + off, length)],
                                  sout),
            pltpu.make_async_copy(dy_v.at[pl.ds(off, length)],
                                  reg_hbm.at[pl.ds(pbase + _N + off, length)],
                                  sout),
            pltpu.make_async_copy(
                dw_v.at[pl.ds(off, length)],
                reg_hbm.at[pl.ds(pbase + 2 * _N + off, length)], sout),
            pltpu.make_async_copy(
                dh_v.at[pl.ds(off, length)],
                reg_hbm.at[pl.ds(pbase + 3 * _N + off, length)], sout),
            pltpu.make_async_copy(cls_v.at[pl.ds(off, length)],
                                  cls_hbm.at[pl.ds(abase + off, length)],
                                  sout),
            pltpu.make_async_copy(clsm_v.at[pl.ds(off, length)],
                                  clsm_hbm.at[pl.ds(abase + off, length)],
                                  sout),
            pltpu.make_async_copy(regm_v.at[pl.ds(off, length)],
                                  regm_hbm.at[pl.ds(abase + off, length)],
                                  sout),
        ]

    tbl_cp = pltpu.make_async_copy(gt_hbm, tbl_v, stbl)
    tbl_cp.start()
    sems = (s0, s1)
    for k in range(_NB):
        for c in in_copies(k * _BLK, _BLK, sems[k]):
            c.start()

    @pl.when(has_extra)
    def _():
        for c in in_copies(n_main, 16, sx):
            c.start()

    lanes = lax.iota(jnp.int32, 16)
    tbl_cp.wait()
    tbase = b * (_M * 5)
    for j in range(_TROWS // 16):
        r = jnp.minimum(j * 16 + lanes, _M - 1) * 5 + tbase
        gx1 = plsc.load_gather(tbl_v, [r])
        gy1 = plsc.load_gather(tbl_v, [r + 1])
        gx2 = plsc.load_gather(tbl_v, [r + 2])
        gy2 = plsc.load_gather(tbl_v, [r + 3])
        gcl = plsc.load_gather(tbl_v, [r + 4])
        sl = pl.ds(j * 16, 16)
        tt_v[sl] = gx1 + gx2
        tt_v[pl.ds(_TROWS + j * 16, 16)] = gy1 + gy2
        tt_v[pl.ds(2 * _TROWS + j * 16, 16)] = _ln(
            jnp.maximum(gx2 - gx1, 1e-3))
        tt_v[pl.ds(3 * _TROWS + j * 16, 16)] = _ln(
            jnp.maximum(gy2 - gy1, 1e-3))
        tt_v[pl.ds(4 * _TROWS + j * 16, 16)] = gcl

    def one_vec(i):
        al = i * 16
        sl = pl.ds(al, 16)
        idx16 = idx_v[sl]
        gsx = plsc.load_gather(tt_v, [idx16])
        gsy = plsc.load_gather(tt_v, [idx16 + _TROWS])
        lgw = plsc.load_gather(tt_v, [idx16 + 2 * _TROWS])
        lgh = plsc.load_gather(tt_v, [idx16 + 3 * _TROWS])
        gcl = plsc.load_gather(tt_v, [idx16 + 4 * _TROWS])

        x1 = x1_v[sl]
        y1 = y1_v[sl]
        x2 = x2_v[sl]
        y2 = y2_v[sl]
        bw = jnp.maximum(x2 - x1, 1e-3)
        bh = jnp.maximum(y2 - y1, 1e-3)
        dx_v[sl] = (gsx - (x1 + x2)) * 0.5 / bw
        dy_v[sl] = (gsy - (y1 + y2)) * 0.5 / bh
        dw_v[sl] = lgw - _ln(bw)
        dh_v[sl] = lgh - _ln(bh)

        flag = flag_v[sl]
        cls = jnp.where(flag == 0, 0.0, gcl)
        regm_v[sl] = jnp.where((flag > 0) & (cls > 0.0), 1, 0)
        cls = jnp.where(flag < 0, -jnp.abs(cls), cls)
        cls_v[sl] = cls
        clsm_v[sl] = jnp.where(cls >= 0.0, 1, 0)

    def loop_body(i, carry):
        one_vec(i)
        return carry

    for k in range(_NB):
        for c in in_copies(k * _BLK, _BLK, sems[k]):
            c.wait()
        lax.fori_loop(k * _BV, (k + 1) * _BV, loop_body, 0)
        for c in out_copies(k * _BLK, _BLK):
            c.start()

    @pl.when(has_extra)
    def _():
        for c in in_copies(n_main, 16, sx):
            c.wait()
        one_vec(jnp.int32(_MAIN))
        for c in out_copies(n_main, 16):
            c.start()

    for k in range(_NB):
        for c in out_copies(k * _BLK, _BLK):
            c.wait()

    @pl.when(has_extra)
    def _():
        for c in out_copies(n_main, 16):
            c.wait()


@jax.jit
def _run(boxes_pl, gt_flat, flag_f, idx_f):
    call = pl.kernel(
        _body,
        out_type=(
            jax.ShapeDtypeStruct((_B * _N,), jnp.float32),
            jax.ShapeDtypeStruct((_B * _N,), jnp.int32),
            jax.ShapeDtypeStruct((_B * 4 * _N,), jnp.float32),
            jax.ShapeDtypeStruct((_B * _N,), jnp.int32),
        ),
        mesh=plsc.VectorSubcoreMesh(core_axis_name="c", subcore_axis_name="s"),
        compiler_params=pltpu.CompilerParams(needs_layout_passes=False, allow_input_fusion=[True, True, True, True]),
        scratch_types=[
            pltpu.VMEM((_B * _M * 5,), jnp.float32),
            pltpu.VMEM((5 * _TROWS,), jnp.float32),
            pltpu.VMEM((_CHUNK,), jnp.float32),
            pltpu.VMEM((_CHUNK,), jnp.float32),
            pltpu.VMEM((_CHUNK,), jnp.float32),
            pltpu.VMEM((_CHUNK,), jnp.float32),
            pltpu.VMEM((_CHUNK,), jnp.int32),
            pltpu.VMEM((_CHUNK,), jnp.int32),
            pltpu.VMEM((_CHUNK,), jnp.float32),
            pltpu.VMEM((_CHUNK,), jnp.int32),
            pltpu.VMEM((_CHUNK,), jnp.float32),
            pltpu.VMEM((_CHUNK,), jnp.float32),
            pltpu.VMEM((_CHUNK,), jnp.float32),
            pltpu.VMEM((_CHUNK,), jnp.float32),
            pltpu.VMEM((_CHUNK,), jnp.int32),
            pltpu.SemaphoreType.DMA,
            pltpu.SemaphoreType.DMA,
            pltpu.SemaphoreType.DMA,
            pltpu.SemaphoreType.DMA,
            pltpu.SemaphoreType.DMA,
        ],
    )
    return call(boxes_pl, gt_flat, flag_f, idx_f)


def kernel(boxes, gt_boxes, match_pos_flag, match_gt_id):
    boxes_pl = boxes.transpose(0, 2, 1).reshape(-1)
    cls, clsm, reg, regm = _run(boxes_pl, gt_boxes.reshape(-1),
                                match_pos_flag.reshape(-1),
                                match_gt_id.reshape(-1))
    cls_label = cls.reshape(_B, _N, 1)
    cls_label_mask = clsm.astype(jnp.bool_).reshape(_B, _N, 1)
    reg_label = reg.reshape(_B, 4, _N).transpose(0, 2, 1)
    reg_label_mask = jnp.broadcast_to(
        regm.astype(jnp.bool_).reshape(_B, _N, 1), (_B, _N, 4))
    return cls_label, cls_label_mask, reg_label, reg_label_mask


# speedup vs baseline: 1.0316x; 1.0021x over previous
import jax
import jax.numpy as jnp
from jax import lax
from jax.experimental import pallas as pl
from jax.experimental.pallas import tpu as pltpu
from jax.experimental.pallas import tpu_sc as plsc

_B, _N, _M = 8, 20000, 100
_VPB = _N // 16
_MAIN = _VPB // 4
_NB = 2
_BV = _MAIN // _NB
_BLK = _BV * 16
_CHUNK = (_MAIN + 1) * 16
_TROWS = 112

_LN2 = 0.6931471805599453
_P5 = (0.04392863, -0.40947559, 1.61017755, -3.52021884, 5.06975632,
       -2.79415368)


def _ln(x):
    bits = lax.bitcast_convert_type(x, jnp.int32)
    e = ((bits >> 23) & 0xFF) - 127
    m = lax.bitcast_convert_type((bits & 0x007FFFFF) | 0x3F800000,
                                 jnp.float32)
    p = jnp.float32(_P5[0])
    for c in _P5[1:]:
        p = p * m + c
    return (e.astype(jnp.float32) + p) * _LN2


def _body(boxes_hbm, gt_hbm, flag_hbm, idx_hbm,
          cls_hbm, clsm_hbm, reg_hbm, regm_hbm,
          tbl_v, tt_v, x1_v, y1_v, x2_v, y2_v, flag_v, idx_v,
          cls_v, clsm_v, dx_v, dy_v, dw_v, dh_v, regm_v,
          stbl, s0, s1, sx, sout):
    nc = 2
    w = lax.axis_index("s") * nc + lax.axis_index("c")
    b = w // 4
    q = w % 4
    has_extra = q < 2
    n0 = (q * 313 - jnp.maximum(q - 2, 0)) * 16
    pbase = b * 4 * _N + n0
    abase = b * _N + n0
    n_main = _MAIN * 16

    def in_copies(off, length, sem):
        return [
            pltpu.make_async_copy(boxes_hbm.at[pl.ds(pbase + off, length)],
                                  x1_v.at[pl.ds(off, length)], sem),
            pltpu.make_async_copy(
                boxes_hbm.at[pl.ds(pbase + _N + off, length)],
                y1_v.at[pl.ds(off, length)], sem),
            pltpu.make_async_copy(
                boxes_hbm.at[pl.ds(pbase + 2 * _N + off, length)],
                x2_v.at[pl.ds(off, length)], sem),
            pltpu.make_async_copy(
                boxes_hbm.at[pl.ds(pbase + 3 * _N + off, length)],
                y2_v.at[pl.ds(off, length)], sem),
            pltpu.make_async_copy(flag_hbm.at[pl.ds(abase + off, length)],
                                  flag_v.at[pl.ds(off, length)], sem),
            pltpu.make_async_copy(idx_hbm.at[pl.ds(abase + off, length)],
                                  idx_v.at[pl.ds(off, length)], sem),
        ]

    def out_copies(off, length):
        return [
            pltpu.make_async_copy(dx_v.at[pl.ds(off, length)],
                                  reg_hbm.at[pl.ds(pbase + off, length)],
                                  sout),
            pltpu.make_async_copy(dy_v.at[pl.ds(off, length)],
                                  reg_hbm.at[pl.ds(pbase + _N + off, length)],
                                  sout),
            pltpu.make_async_copy(
                dw_v.at[pl.ds(off, length)],
                reg_hbm.at[pl.ds(pbase + 2 * _N + off, length)], sout),
            pltpu.make_async_copy(
                dh_v.at[pl.ds(off, length)],
                reg_hbm.at[pl.ds(pbase + 3 * _N + off, length)], sout),
            pltpu.make_async_copy(cls_v.at[pl.ds(off, length)],
                                  cls_hbm.at[pl.ds(abase + off, length)],
                                  sout),
            pltpu.make_async_copy(clsm_v.at[pl.ds(off, length)],
                                  clsm_hbm.at[pl.ds(abase + off, length)],
                                  sout),
            pltpu.make_async_copy(regm_v.at[pl.ds(off, length)],
                                  regm_hbm.at[pl.ds(abase + off, length)],
                                  sout),
        ]

    tbl_cp = pltpu.make_async_copy(gt_hbm, tbl_v, stbl)
    tbl_cp.start()
    sems = (s0, s1)
    for k in range(_NB):
        for c in in_copies(k * _BLK, _BLK, sems[k]):
            c.start()

    @pl.when(has_extra)
    def _():
        for c in in_copies(n_main, 16, sx):
            c.start()

    lanes = lax.iota(jnp.int32, 16)
    tbl_cp.wait()
    tbase = b * (_M * 5)
    for j in range(_TROWS // 16):
        r = jnp.minimum(j * 16 + lanes, _M - 1) * 5 + tbase
        gx1 = plsc.load_gather(tbl_v, [r])
        gy1 = plsc.load_gather(tbl_v, [r + 1])
        gx2 = plsc.load_gather(tbl_v, [r + 2])
        gy2 = plsc.load_gather(tbl_v, [r + 3])
        gcl = plsc.load_gather(tbl_v, [r + 4])
        sl = pl.ds(j * 16, 16)
        tt_v[sl] = gx1 + gx2
        tt_v[pl.ds(_TROWS + j * 16, 16)] = gy1 + gy2
        tt_v[pl.ds(2 * _TROWS + j * 16, 16)] = _ln(
            jnp.maximum(gx2 - gx1, 1e-3))
        tt_v[pl.ds(3 * _TROWS + j * 16, 16)] = _ln(
            jnp.maximum(gy2 - gy1, 1e-3))
        tt_v[pl.ds(4 * _TROWS + j * 16, 16)] = gcl

    def one_vec(i):
        al = i * 16
        sl = pl.ds(al, 16)
        idx16 = idx_v[sl]
        gsx = plsc.load_gather(tt_v, [idx16])
        gsy = plsc.load_gather(tt_v, [idx16 + _TROWS])
        lgw = plsc.load_gather(tt_v, [idx16 + 2 * _TROWS])
        lgh = plsc.load_gather(tt_v, [idx16 + 3 * _TROWS])
        gcl = plsc.load_gather(tt_v, [idx16 + 4 * _TROWS])

        x1 = x1_v[sl]
        y1 = y1_v[sl]
        x2 = x2_v[sl]
        y2 = y2_v[sl]
        bw = jnp.maximum(x2 - x1, 1e-3)
        bh = jnp.maximum(y2 - y1, 1e-3)
        dx_v[sl] = (gsx - (x1 + x2)) * 0.5 / bw
        dy_v[sl] = (gsy - (y1 + y2)) * 0.5 / bh
        dw_v[sl] = lgw - _ln(bw)
        dh_v[sl] = lgh - _ln(bh)

        flag = flag_v[sl]
        cls = jnp.where(flag == 0, 0.0, gcl)
        regm_v[sl] = jnp.where((flag > 0) & (cls > 0.0), 1, 0)
        cls = jnp.where(flag < 0, -jnp.abs(cls), cls)
        cls_v[sl] = cls
        clsm_v[sl] = jnp.where(cls >= 0.0, 1, 0)

    def loop_body(i, carry):
        one_vec(i)
        return carry

    for k in range(_NB):
        for c in in_copies(k * _BLK, _BLK, sems[k]):
            c.wait()
        lax.fori_loop(k * _BV, (k + 1) * _BV, loop_body, 0)
        for c in out_copies(k * _BLK, _BLK):
            c.start()

    @pl.when(has_extra)
    def _():
        for c in in_copies(n_main, 16, sx):
            c.wait()
        one_vec(jnp.int32(_MAIN))
        for c in out_copies(n_main, 16):
            c.start()

    for k in range(_NB):
        for c in out_copies(k * _BLK, _BLK):
            c.wait()

    @pl.when(has_extra)
    def _():
        for c in out_copies(n_main, 16):
            c.wait()


@jax.jit
def _run(boxes_pl, gt_flat, flag_f, idx_f):
    call = pl.kernel(
        _body,
        out_type=(
            jax.ShapeDtypeStruct((_B * _N,), jnp.float32),
            jax.ShapeDtypeStruct((_B * _N,), jnp.int32),
            jax.ShapeDtypeStruct((_B * 4 * _N,), jnp.float32),
            jax.ShapeDtypeStruct((_B * _N,), jnp.int32),
        ),
        mesh=plsc.VectorSubcoreMesh(core_axis_name="c", subcore_axis_name="s"),
        compiler_params=pltpu.CompilerParams(needs_layout_passes=False),
        scratch_types=[
            pltpu.VMEM((_B * _M * 5,), jnp.float32),
            pltpu.VMEM((5 * _TROWS,), jnp.float32),
            pltpu.VMEM((_CHUNK,), jnp.float32),
            pltpu.VMEM((_CHUNK,), jnp.float32),
            pltpu.VMEM((_CHUNK,), jnp.float32),
            pltpu.VMEM((_CHUNK,), jnp.float32),
            pltpu.VMEM((_CHUNK,), jnp.int32),
            pltpu.VMEM((_CHUNK,), jnp.int32),
            pltpu.VMEM((_CHUNK,), jnp.float32),
            pltpu.VMEM((_CHUNK,), jnp.int32),
            pltpu.VMEM((_CHUNK,), jnp.float32),
            pltpu.VMEM((_CHUNK,), jnp.float32),
            pltpu.VMEM((_CHUNK,), jnp.float32),
            pltpu.VMEM((_CHUNK,), jnp.float32),
            pltpu.VMEM((_CHUNK,), jnp.int32),
            pltpu.SemaphoreType.DMA,
            pltpu.SemaphoreType.DMA,
            pltpu.SemaphoreType.DMA,
            pltpu.SemaphoreType.DMA,
            pltpu.SemaphoreType.DMA,
        ],
    )
    return call(boxes_pl, gt_flat, flag_f, idx_f)


def kernel(boxes, gt_boxes, match_pos_flag, match_gt_id):
    boxes_pl = boxes.transpose(0, 2, 1).reshape(-1)
    cls, clsm, reg, regm = _run(boxes_pl, gt_boxes.reshape(-1),
                                match_pos_flag.reshape(-1),
                                match_gt_id.reshape(-1))
    cls_label = cls.reshape(_B, _N, 1)
    cls_label_mask = clsm.astype(jnp.bool_).reshape(_B, _N, 1)
    reg_label = reg.reshape(_B, 4, _N).transpose(0, 2, 1)
    reg_label_mask = jnp.broadcast_to(
        regm.astype(jnp.bool_).reshape(_B, _N, 1), (_B, _N, 4))
    return cls_label, cls_label_mask, reg_label, reg_label_mask
